# baseline (speedup 1.0000x reference)
import functools

import jax
import jax.numpy as jnp
from jax import lax
from jax.experimental import pallas as pl
from jax.experimental.pallas import tpu as pltpu
from jax.experimental.pallas import tpu_sc as plsc

N = 16384
D = 64
K = 5
NC = 2
NS = 16
NW = NC * NS
R = N // NW
CH = 128
NCHUNK = R // CH
L = 16
NL = D // L

SC_OUT = N * (1 + K) * L

_mesh = plsc.VectorSubcoreMesh(core_axis_name="c", subcore_axis_name="s")


@functools.partial(
    pl.kernel,
    mesh=_mesh,
    out_type=jax.ShapeDtypeStruct((SC_OUT,), jnp.float32),
    scratch_types=[
        pltpu.VMEM((CH,), jnp.int32),
        pltpu.VMEM((K, CH), jnp.int32),
        pltpu.VMEM((CH, D), jnp.float32),
        pltpu.VMEM((K * CH, D), jnp.float32),
        pltpu.VMEM((CH, D), jnp.float32),
        pltpu.VMEM((CH * L,), jnp.float32),
        pltpu.VMEM((K * CH * L,), jnp.float32),
        pltpu.SemaphoreType.DMA,
    ],
    compiler_params=pltpu.CompilerParams(use_tc_tiling_on_sc=False),
)
def _sc_scores(table, lab_idx, neg_idx, hid, out,
               pidx_v, nidx_v, lab_v, neg_v, hid_v, ps_v, ns_v, sem):
    w = lax.axis_index("s") * NC + lax.axis_index("c")
    for c in range(NCHUNK):
        base = w * R + c * CH
        pltpu.sync_copy(lab_idx.at[pl.ds(base, CH)], pidx_v)
        for k in range(K):
            pltpu.sync_copy(neg_idx.at[pl.ds(base * K + k * CH, CH)],
                            nidx_v.at[k])
        copies = [pltpu.async_copy(table.at[pidx_v], lab_v, sem)]
        for k in range(K):
            copies.append(pltpu.async_copy(table.at[nidx_v.at[k]],
                                           neg_v.at[pl.ds(k * CH, CH)], sem))
        copies.append(pltpu.async_copy(hid.at[pl.ds(base, CH)], hid_v, sem))
        for cp in copies:
            cp.wait()

        def body(i, _):
            hs = [hid_v[i, pl.ds(t * L, L)] for t in range(NL)]
            acc = hs[0] * lab_v[i, pl.ds(0, L)]
            for t in range(1, NL):
                acc = acc + hs[t] * lab_v[i, pl.ds(t * L, L)]
            ps_v[pl.ds(i * L, L)] = acc
            for k in range(K):
                j = i * K + k
                acc = hs[0] * neg_v[j, pl.ds(0, L)]
                for t in range(1, NL):
                    acc = acc + hs[t] * neg_v[j, pl.ds(t * L, L)]
                ns_v[pl.ds(j * L, L)] = acc
            return 0

        lax.fori_loop(0, CH, body, 0)
        pltpu.sync_copy(ps_v, out.at[pl.ds(base * L, CH * L)])
        pltpu.sync_copy(ns_v, out.at[pl.ds((N + base * K) * L, K * CH * L)])


TC_ROWS = SC_OUT // 128
TC_BLK = 2048


def _tc_body(s_ref, o_ref):
    b = pl.program_id(0)
    pos = b == 0
    sgn = jnp.where(pos, 1.0, -1.0)
    wgt = jnp.where(pos, 1.0, 1.0 / K)
    x = s_ref[...]
    acc = jnp.float32(0.0)
    for g in range(128 // L):
        score = jnp.sum(x[:, g * L:(g + 1) * L], axis=1, keepdims=True)
        y = sgn * score
        ls = jnp.minimum(y, 0.0) - jnp.log1p(jnp.exp(-jnp.abs(y)))
        acc = acc + jnp.sum(ls)

    @pl.when(b == 0)
    def _():
        o_ref[0, 0] = 0.0

    o_ref[0, 0] = o_ref[0, 0] - wgt * acc


_tc_loss = pl.pallas_call(
    _tc_body,
    grid=(TC_ROWS // TC_BLK,),
    in_specs=[pl.BlockSpec((TC_BLK, 128), lambda i: (i, 0))],
    out_specs=pl.BlockSpec(memory_space=pltpu.SMEM, block_shape=(1, 1),
                           index_map=lambda i: (0, 0)),
    out_shape=jax.ShapeDtypeStruct((1, 1), jnp.float32),
    compiler_params=pltpu.CompilerParams(
        dimension_semantics=("arbitrary",)),
)


def kernel(hidden_state, label_idxes, neg_idxes, out_word_emb):
    lab = label_idxes.astype(jnp.int32)
    neg = neg_idxes.astype(jnp.int32)
    partials = _sc_scores(out_word_emb, lab, neg, hidden_state)
    loss = _tc_loss(partials.reshape(TC_ROWS, 128))
    return loss.reshape(())


# speedup vs baseline: 1.0600x; 1.0600x over previous
import functools

import jax
import jax.numpy as jnp
from jax import lax
from jax.experimental import pallas as pl
from jax.experimental.pallas import tpu as pltpu
from jax.experimental.pallas import tpu_sc as plsc

N = 16384
D = 64
DP = 128
K = 5
NC = 2
NS = 16
NW = NC * NS
R = N // NW
CH = 64
NCHUNK = R // CH
L = 16
NL = D // L

SC_OUT = N * (1 + K) * L

_mesh = plsc.VectorSubcoreMesh(core_axis_name="c", subcore_axis_name="s")


@functools.partial(
    pl.kernel,
    mesh=_mesh,
    out_type=jax.ShapeDtypeStruct((SC_OUT,), jnp.float32),
    scratch_types=[
        pltpu.VMEM((CH,), jnp.int32),
        pltpu.VMEM((K, CH), jnp.int32),
        pltpu.VMEM((CH, DP), jnp.float32),
        pltpu.VMEM((K * CH, DP), jnp.float32),
        pltpu.VMEM((CH, DP), jnp.float32),
        pltpu.VMEM((CH * L,), jnp.float32),
        pltpu.VMEM((K * CH * L,), jnp.float32),
        pltpu.SemaphoreType.DMA,
    ],
)
def _sc_scores(table, lab_idx, neg_idx, hid, out,
               pidx_v, nidx_v, lab_v, neg_v, hid_v, ps_v, ns_v, sem):
    w = lax.axis_index("s") * NC + lax.axis_index("c")
    for c in range(NCHUNK):
        base = w * R + c * CH
        pltpu.sync_copy(lab_idx.at[pl.ds(base, CH)], pidx_v)
        for k in range(K):
            pltpu.sync_copy(neg_idx.at[pl.ds(base * K + k * CH, CH)],
                            nidx_v.at[k])
        copies = [pltpu.async_copy(table.at[pidx_v], lab_v, sem)]
        for k in range(K):
            copies.append(pltpu.async_copy(table.at[nidx_v.at[k]],
                                           neg_v.at[pl.ds(k * CH, CH)], sem))
        copies.append(pltpu.async_copy(hid.at[pl.ds(base, CH)], hid_v, sem))
        for cp in copies:
            cp.wait()

        def body(i, _):
            hs = [hid_v[i, pl.ds(t * L, L)] for t in range(NL)]
            acc = hs[0] * lab_v[i, pl.ds(0, L)]
            for t in range(1, NL):
                acc = acc + hs[t] * lab_v[i, pl.ds(t * L, L)]
            ps_v[pl.ds(i * L, L)] = acc
            for k in range(K):
                j = i * K + k
                acc = hs[0] * neg_v[j, pl.ds(0, L)]
                for t in range(1, NL):
                    acc = acc + hs[t] * neg_v[j, pl.ds(t * L, L)]
                ns_v[pl.ds(j * L, L)] = acc
            return 0

        lax.fori_loop(0, CH, body, 0)
        pltpu.sync_copy(ps_v, out.at[pl.ds(base * L, CH * L)])
        pltpu.sync_copy(ns_v, out.at[pl.ds((N + base * K) * L, K * CH * L)])


TC_ROWS = SC_OUT // 128
TC_BLK = 2048


def _tc_body(s_ref, o_ref):
    b = pl.program_id(0)
    pos = b == 0
    sgn = jnp.where(pos, 1.0, -1.0)
    wgt = jnp.where(pos, 1.0, 1.0 / K)
    x = s_ref[...]
    acc = jnp.float32(0.0)
    for g in range(128 // L):
        score = jnp.sum(x[:, g * L:(g + 1) * L], axis=1, keepdims=True)
        y = sgn * score
        ls = jnp.minimum(y, 0.0) - jnp.log1p(jnp.exp(-jnp.abs(y)))
        acc = acc + jnp.sum(ls)

    @pl.when(b == 0)
    def _():
        o_ref[0, 0] = 0.0

    o_ref[0, 0] = o_ref[0, 0] - wgt * acc


_tc_loss = pl.pallas_call(
    _tc_body,
    grid=(TC_ROWS // TC_BLK,),
    in_specs=[pl.BlockSpec((TC_BLK, 128), lambda i: (i, 0))],
    out_specs=pl.BlockSpec(memory_space=pltpu.SMEM, block_shape=(1, 1),
                           index_map=lambda i: (0, 0)),
    out_shape=jax.ShapeDtypeStruct((1, 1), jnp.float32),
    compiler_params=pltpu.CompilerParams(
        dimension_semantics=("arbitrary",)),
)


def kernel(hidden_state, label_idxes, neg_idxes, out_word_emb):
    lab = label_idxes.astype(jnp.int32)
    neg = neg_idxes.astype(jnp.int32)
    table_p = jnp.pad(out_word_emb, ((0, 0), (0, DP - D)))
    hid_p = jnp.pad(hidden_state, ((0, 0), (0, DP - D)))
    partials = _sc_scores(table_p, lab, neg, hid_p)
    loss = _tc_loss(partials.reshape(TC_ROWS, 128))
    return loss.reshape(())
